# baseline (speedup 1.0000x reference)
import functools

import jax
import jax.numpy as jnp
from jax import lax
from jax.experimental import pallas as pl
from jax.experimental.pallas import tpu as pltpu
from jax.experimental.pallas import tpu_sc as plsc

_ALPHA_GAIN = (1.0 / (0.01 * 1000000.0)) / 2.0

_NC, _NS = 2, 16
_NW = _NC * _NS
_CHUNK = 16384
_K = 2
_LANES = 16


def _sc_body(x_hbm, o_hbm, in_buf, out_buf, in_sem, out_sem, *, per_w, nch):
    wid = lax.axis_index("s") * _NC + lax.axis_index("c")
    base = wid * per_w

    def start_in(i, s):
        pltpu.make_async_copy(
            x_hbm.at[pl.ds(base + i * _CHUNK, _CHUNK)],
            in_buf.at[s], in_sem.at[s]).start()

    def wait_in(i, s):
        pltpu.make_async_copy(
            x_hbm.at[pl.ds(base + i * _CHUNK, _CHUNK)],
            in_buf.at[s], in_sem.at[s]).wait()

    def start_out(i, s):
        pltpu.make_async_copy(
            out_buf.at[s],
            o_hbm.at[pl.ds(base + i * _CHUNK, _CHUNK)],
            out_sem.at[s]).start()

    def wait_out(i, s):
        pltpu.make_async_copy(
            out_buf.at[s],
            o_hbm.at[pl.ds(base + i * _CHUNK, _CHUNK)],
            out_sem.at[s]).wait()

    for i in range(min(_K, nch)):
        start_in(i, i)
    for i in range(nch):
        s = i % _K
        wait_in(i, s)
        if i >= _K:
            wait_out(i - _K, s)

        @plsc.parallel_loop(0, _CHUNK // (_LANES * 8), unroll=4)
        def _mul(j):
            base_off = j * (_LANES * 8)
            for u in range(8):
                off = base_off + u * _LANES
                out_buf[s, pl.ds(off, _LANES)] = (
                    in_buf[s, pl.ds(off, _LANES)] * _ALPHA_GAIN)

        start_out(i, s)
        if i + _K < nch:
            start_in(i + _K, s)
    for i in range(max(nch - _K, 0), nch):
        wait_out(i, i % _K)


def kernel(t_in, rate_hopping, y_in, inds_surf, inds_mant, dy_surf_gain, dy_surf_loss, inds_r_m2s):
    b, n = dy_surf_gain.shape
    total = b * n
    per_w = total // _NW
    nch = per_w // _CHUNK
    x = dy_surf_gain.reshape(total)
    mesh = plsc.VectorSubcoreMesh(
        core_axis_name="c", subcore_axis_name="s",
        num_cores=_NC, num_subcores=_NS)
    sc_call = pl.kernel(
        functools.partial(_sc_body, per_w=per_w, nch=nch),
        out_type=jax.ShapeDtypeStruct((total,), jnp.float32),
        mesh=mesh,
        scratch_types=[
            pltpu.VMEM((_K, _CHUNK), jnp.float32),
            pltpu.VMEM((_K, _CHUNK), jnp.float32),
            pltpu.SemaphoreType.DMA((_K,)),
            pltpu.SemaphoreType.DMA((_K,)),
        ],
    )
    return sc_call(x).reshape(b, n)


# speedup vs baseline: 5.6896x; 5.6896x over previous
import jax
import jax.numpy as jnp
from jax.experimental import pallas as pl
from jax.experimental.pallas import tpu as pltpu

_LAYER_FACTOR = 1.0 / (0.01 * 1000000.0)
_NUM_ACTIVE_LAYERS = 2.0
_ALPHA_GAIN = _LAYER_FACTOR / _NUM_ACTIVE_LAYERS

_BLOCK_ROWS = 2048


def _scale_body(x_ref, o_ref):
    o_ref[...] = x_ref[...] * _ALPHA_GAIN


def kernel(t_in, rate_hopping, y_in, inds_surf, inds_mant, dy_surf_gain, dy_surf_loss, inds_r_m2s):
    b, n = dy_surf_gain.shape
    grid = (b // _BLOCK_ROWS,)
    return pl.pallas_call(
        _scale_body,
        grid=grid,
        in_specs=[pl.BlockSpec((_BLOCK_ROWS, n), lambda i: (i, 0))],
        out_specs=pl.BlockSpec((_BLOCK_ROWS, n), lambda i: (i, 0)),
        out_shape=jax.ShapeDtypeStruct((b, n), dy_surf_gain.dtype),
        compiler_params=pltpu.CompilerParams(
            dimension_semantics=("parallel",),
        ),
    )(dy_surf_gain)
